# speedup vs baseline: 3.2130x; 3.2130x over previous
import functools

import jax
import jax.numpy as jnp
from jax import lax
from jax.experimental import pallas as pl
from jax.experimental.pallas import tpu as pltpu
from jax.experimental.pallas import tpu_sc as plsc

_D = 128
_NW = 32
_CHUNK = 128


def _gather_body(n_chunks, table_hbm, idx_hbm, out_hbm, idx_v, rows_v, sem):
    b_per_w = n_chunks * _CHUNK
    wid = lax.axis_index("s") * 2 + lax.axis_index("c")
    base = wid * b_per_w
    pltpu.sync_copy(idx_hbm.at[pl.ds(base, b_per_w)], idx_v)

    def chunk(j, carry):
        off = j * _CHUNK
        pltpu.async_copy(
            table_hbm.at[idx_v.at[pl.ds(off, _CHUNK)]], rows_v, sem
        ).wait()
        pltpu.sync_copy(rows_v, out_hbm.at[pl.ds(base + off, _CHUNK)])
        return carry

    lax.fori_loop(0, n_chunks, chunk, 0)


@functools.partial(jax.jit, static_argnames=("n_rows",))
def _gather(idx_flat, pattern, n_rows):
    b_per_w = n_rows // _NW
    n_chunks = b_per_w // _CHUNK
    run = pl.kernel(
        functools.partial(_gather_body, n_chunks),
        out_type=jax.ShapeDtypeStruct((n_rows, _D), jnp.float32),
        mesh=plsc.VectorSubcoreMesh(core_axis_name="c", subcore_axis_name="s"),
        scratch_types=[
            pltpu.VMEM((b_per_w,), jnp.int32),
            pltpu.VMEM((_CHUNK, _D), jnp.float32),
            pltpu.SemaphoreType.DMA,
        ],
    )
    return run(pattern, idx_flat)


def kernel(input, pattern):
    b, t = input.shape
    idx_flat = input.reshape(-1).astype(jnp.int32)
    out = _gather(idx_flat, pattern, b * t)
    return out.reshape(b, t, _D)


# speedup vs baseline: 3.5624x; 1.1088x over previous
import functools

import jax
import jax.numpy as jnp
from jax import lax
from jax.experimental import pallas as pl
from jax.experimental.pallas import tpu as pltpu
from jax.experimental.pallas import tpu_sc as plsc

_D = 128
_NW = 32
_CHUNK = 128


_K = 5


def _gather_body(n_groups, table_hbm, idx_hbm, out_hbm, idx_v, rows_v,
                 sem_in, sem_out):
    b_per_w = n_groups * _K * _CHUNK
    wid = lax.axis_index("s") * 2 + lax.axis_index("c")
    base = wid * b_per_w
    pltpu.sync_copy(idx_hbm.at[pl.ds(base, b_per_w)], idx_v)

    def group(g, carry):
        g0 = g * _K * _CHUNK

        @pl.when(g > 0)
        def _():
            for b in range(_K):
                pltpu.make_async_copy(
                    rows_v.at[b], out_hbm.at[pl.ds(base, _CHUNK)], sem_out
                ).wait()

        for b in range(_K):
            pltpu.async_copy(
                table_hbm.at[idx_v.at[pl.ds(g0 + b * _CHUNK, _CHUNK)]],
                rows_v.at[b], sem_in,
            )
        for b in range(_K):
            pltpu.make_async_copy(
                table_hbm.at[idx_v.at[pl.ds(g0, _CHUNK)]],
                rows_v.at[b], sem_in,
            ).wait()
        for b in range(_K):
            pltpu.async_copy(
                rows_v.at[b],
                out_hbm.at[pl.ds(base + g0 + b * _CHUNK, _CHUNK)],
                sem_out,
            )
        return carry

    lax.fori_loop(0, n_groups, group, 0)
    for b in range(_K):
        pltpu.make_async_copy(
            rows_v.at[b], out_hbm.at[pl.ds(base, _CHUNK)], sem_out
        ).wait()


@functools.partial(jax.jit, static_argnames=("n_rows",))
def _gather(idx_flat, pattern, n_rows):
    b_per_w = n_rows // _NW
    n_groups = b_per_w // (_K * _CHUNK)
    run = pl.kernel(
        functools.partial(_gather_body, n_groups),
        out_type=jax.ShapeDtypeStruct((n_rows, _D), jnp.float32),
        mesh=plsc.VectorSubcoreMesh(core_axis_name="c", subcore_axis_name="s"),
        scratch_types=[
            pltpu.VMEM((b_per_w,), jnp.int32),
            pltpu.VMEM((_K, _CHUNK, _D), jnp.float32),
            pltpu.SemaphoreType.DMA,
            pltpu.SemaphoreType.DMA,
        ],
    )
    return run(pattern, idx_flat)


def kernel(input, pattern):
    b, t = input.shape
    idx_flat = input.reshape(-1).astype(jnp.int32)
    out = _gather(idx_flat, pattern, b * t)
    return out.reshape(b, t, _D)
